# Initial kernel scaffold; baseline (speedup 1.0000x reference)
import jax
import jax.numpy as jnp
from jax.experimental import pallas as pl


def kernel(y, Win, b_in, ln_g, ln_b, We1, be1, We2, be2, Wn1, bn1, Wn2, bn2, Wout, bout):
    raise NotImplementedError("write your pallas kernel here")


# speedup vs baseline: 14.6855x; 14.6855x over previous
import numpy as np
import jax
import jax.numpy as jnp
from jax.experimental import pallas as pl
from jax.experimental.pallas import tpu as pltpu

_GRID = 8
_SPACING = 2.0
_RADIUS = 3.5
_NGRID = _GRID ** 3
_PAD = 80


def _stencil_constants():
    r = int(np.floor(_RADIUS / _SPACING)) + 1
    offsets = []
    for dx in range(-r, r + 1):
        for dy in range(-r, r + 1):
            for dz in range(-r, r + 1):
                if dx == 0 and dy == 0 and dz == 0:
                    continue
                d = _SPACING * np.sqrt(dx * dx + dy * dy + dz * dz)
                if d <= _RADIUS + 1e-8:
                    k = dx * _GRID * _GRID + dy * _GRID + dz
                    offsets.append((dx, dy, dz, float(d), k))
    idx = np.arange(_NGRID)
    ix, iy, iz = idx // (_GRID * _GRID), (idx // _GRID) % _GRID, idx % _GRID
    valid = np.zeros((len(offsets), _NGRID), np.float32)
    for i, (dx, dy, dz, _, _) in enumerate(offsets):
        valid[i] = ((ix + dx >= 0) & (ix + dx < _GRID) &
                    (iy + dy >= 0) & (iy + dy < _GRID) &
                    (iz + dz >= 0) & (iz + dz < _GRID)).astype(np.float32)
    denom = np.maximum(valid.sum(0), 1.0)
    scale = valid / denom[None, :]
    return offsets, scale


_OFFSETS, _SCALE = _stencil_constants()
_NOFF = len(_OFFSETS)


def _silu(v):
    return v * jax.nn.sigmoid(v)


def _gnn_kernel(y_ref, win_ref, bin_ref, lng_ref, lnb_ref,
                we1a_ref, we1b_ref, we1c_ref, be1_ref, we2_ref, be2_ref,
                wn1a_ref, wn1b_ref, bn1_ref, wn2_ref, bn2_ref,
                wout_ref, bout_ref, scale_ref, out_ref,
                apad_ref, m1_ref):
    layers = lng_ref.shape[0]
    apad_ref[0:_PAD, :] = jnp.zeros((_PAD, 64), jnp.float32)
    apad_ref[_PAD + _NGRID:, :] = jnp.zeros_like(apad_ref[_PAD + _NGRID:, :])

    h = jnp.dot(y_ref[0], win_ref[...],
                preferred_element_type=jnp.float32) + bin_ref[...]
    for l in range(layers):
        mu = jnp.mean(h, axis=1, keepdims=True)
        var = jnp.mean((h - mu) ** 2, axis=1, keepdims=True)
        hn = (h - mu) * jax.lax.rsqrt(var + 1e-5) * lng_ref[l] + lnb_ref[l]
        a = jnp.dot(hn, we1a_ref[l], preferred_element_type=jnp.float32)
        b = jnp.dot(hn, we1b_ref[l], preferred_element_type=jnp.float32)
        apad_ref[_PAD:_PAD + _NGRID, :] = a
        for i, (_, _, _, dist, k) in enumerate(_OFFSETS):
            c = dist * we1c_ref[l] + be1_ref[l]
            a_sh = apad_ref[_PAD + k:_PAD + k + _NGRID, :]
            m1_ref[i * _NGRID:(i + 1) * _NGRID, :] = _silu(a_sh + b + c)
        m2 = _silu(jnp.dot(m1_ref[...], we2_ref[l],
                           preferred_element_type=jnp.float32) + be2_ref[l])
        m2 = m2 * scale_ref[...]
        magg = jnp.zeros((_NGRID, 64), jnp.float32)
        for i in range(_NOFF):
            magg = magg + m2[i * _NGRID:(i + 1) * _NGRID, :]
        nd = _silu(jnp.dot(hn, wn1a_ref[l], preferred_element_type=jnp.float32)
                   + jnp.dot(magg, wn1b_ref[l],
                             preferred_element_type=jnp.float32)
                   + bn1_ref[l])
        h = hn + jnp.dot(nd, wn2_ref[l],
                         preferred_element_type=jnp.float32) + bn2_ref[l]
    out_ref[0] = jnp.dot(h, wout_ref[...],
                         preferred_element_type=jnp.float32) + bout_ref[...]


def kernel(y, Win, b_in, ln_g, ln_b, We1, be1, We2, be2,
           Wn1, bn1, Wn2, bn2, Wout, bout):
    bsz, ngrid, code = y.shape
    hid = Win.shape[1]
    layers = ln_g.shape[0]

    scale = jnp.asarray(np.repeat(_SCALE.reshape(-1, 1), hid, axis=1))

    full = lambda arr: pl.BlockSpec(arr.shape, lambda bb: (0,) * arr.ndim)
    args = (
        Win, b_in.reshape(1, hid), ln_g[:, None, :], ln_b[:, None, :],
        We1[:, :hid, :], We1[:, hid:2 * hid, :], We1[:, 2 * hid:, :],
        be1[:, None, :], We2, be2[:, None, :],
        Wn1[:, :hid, :], Wn1[:, hid:, :], bn1[:, None, :],
        Wn2, bn2[:, None, :], Wout, bout.reshape(1, code), scale,
    )
    out = pl.pallas_call(
        _gnn_kernel,
        grid=(bsz,),
        in_specs=[pl.BlockSpec((1, ngrid, code), lambda bb: (bb, 0, 0))]
                 + [full(a) for a in args],
        out_specs=pl.BlockSpec((1, ngrid, code), lambda bb: (bb, 0, 0)),
        out_shape=jax.ShapeDtypeStruct((bsz, ngrid, code), jnp.float32),
        scratch_shapes=[
            pltpu.VMEM((2 * _PAD + ngrid, hid), jnp.float32),
            pltpu.VMEM((_NOFF * ngrid, hid), jnp.float32),
        ],
        compiler_params=pltpu.CompilerParams(
            dimension_semantics=("parallel",)),
    )(y, *args)
    return out


# speedup vs baseline: 23.1297x; 1.5750x over previous
import numpy as np
import jax
import jax.numpy as jnp
from jax.experimental import pallas as pl
from jax.experimental.pallas import tpu as pltpu

_GRID = 8
_SPACING = 2.0
_RADIUS = 3.5
_NGRID = _GRID ** 3
_PAD = 80


def _stencil_constants():
    r = int(np.floor(_RADIUS / _SPACING)) + 1
    offsets = []
    for dx in range(-r, r + 1):
        for dy in range(-r, r + 1):
            for dz in range(-r, r + 1):
                if dx == 0 and dy == 0 and dz == 0:
                    continue
                d = _SPACING * np.sqrt(dx * dx + dy * dy + dz * dz)
                if d <= _RADIUS + 1e-8:
                    offsets.append((dx, dy, dz, float(d)))
    idx = np.arange(_NGRID)
    ix, iy, iz = idx // (_GRID * _GRID), (idx // _GRID) % _GRID, idx % _GRID
    valid = np.zeros((len(offsets), _NGRID), np.float32)
    for i, (dx, dy, dz, _) in enumerate(offsets):
        valid[i] = ((ix + dx >= 0) & (ix + dx < _GRID) &
                    (iy + dy >= 0) & (iy + dy < _GRID) &
                    (iz + dz >= 0) & (iz + dz < _GRID)).astype(np.float32)
    denom = np.maximum(valid.sum(0), 1.0)
    scale = valid / denom[None, :]
    return offsets, scale


_OFFSETS, _SCALE = _stencil_constants()
_NOFF = len(_OFFSETS)
_DISTS = sorted({d for (_, _, _, d) in _OFFSETS})


def _silu(v):
    u = 0.5 * v
    return u * jnp.tanh(u) + u


def _gnn_kernel(y_ref, win_ref, bin_ref, lng_ref, lnb_ref,
                we1a_ref, we1b_ref, we1c_ref, be1_ref, we2_ref, be2_ref,
                wn1a_ref, wn1b_ref, bn1_ref, wn2_ref, bn2_ref,
                wout_ref, bout_ref, scale_ref, out_ref,
                az0_ref, azp_ref, azm_ref):
    layers = lng_ref.shape[0]
    n = _NGRID
    az0_ref[0:_PAD, :] = jnp.zeros((_PAD, 64), jnp.float32)
    az0_ref[_PAD + n:, :] = jnp.zeros_like(az0_ref[_PAD + n:, :])
    azp_ref[...] = jnp.zeros_like(azp_ref)
    azm_ref[...] = jnp.zeros_like(azm_ref)

    h = jnp.dot(y_ref[0], win_ref[...],
                preferred_element_type=jnp.float32) + bin_ref[...]
    for l in range(layers):
        mu = jnp.mean(h, axis=1, keepdims=True)
        var = jnp.mean((h - mu) ** 2, axis=1, keepdims=True)
        hn = (h - mu) * jax.lax.rsqrt(var + 1e-5) * lng_ref[l] + lnb_ref[l]
        a = jnp.dot(hn, we1a_ref[l], preferred_element_type=jnp.float32)
        b = jnp.dot(hn, we1b_ref[l], preferred_element_type=jnp.float32)
        az0_ref[_PAD:_PAD + n, :] = a
        azp_ref[0:664, :] = az0_ref[1:665, :]
        azm_ref[8:672, :] = az0_ref[7:671, :]
        bc = {d: b + (d * we1c_ref[l] + be1_ref[l]) for d in _DISTS}
        m1s = []
        for (dx, dy, dz, dist) in _OFFSETS:
            base = azp_ref if dz == 1 else (azm_ref if dz == -1 else az0_ref)
            kk = _PAD + dx * 64 + dy * 8
            m1s.append(_silu(base[kk:kk + n, :] + bc[dist]))
        m1 = jnp.concatenate(m1s, axis=0)
        m2 = _silu(jnp.dot(m1, we2_ref[l],
                           preferred_element_type=jnp.float32) + be2_ref[l])
        magg = jnp.zeros((n, 64), jnp.float32)
        for i in range(_NOFF):
            sl = slice(i * n, (i + 1) * n)
            magg = magg + m2[sl] * scale_ref[sl]
        nd = _silu(jnp.dot(hn, wn1a_ref[l], preferred_element_type=jnp.float32)
                   + jnp.dot(magg, wn1b_ref[l],
                             preferred_element_type=jnp.float32)
                   + bn1_ref[l])
        h = hn + jnp.dot(nd, wn2_ref[l],
                         preferred_element_type=jnp.float32) + bn2_ref[l]
    out_ref[0] = jnp.dot(h, wout_ref[...],
                         preferred_element_type=jnp.float32) + bout_ref[...]


def kernel(y, Win, b_in, ln_g, ln_b, We1, be1, We2, be2,
           Wn1, bn1, Wn2, bn2, Wout, bout):
    bsz, ngrid, code = y.shape
    hid = Win.shape[1]

    scale = jnp.asarray(np.repeat(_SCALE.reshape(-1, 1), hid, axis=1))

    full = lambda arr: pl.BlockSpec(arr.shape, lambda bb: (0,) * arr.ndim)
    args = (
        Win, b_in.reshape(1, hid), ln_g[:, None, :], ln_b[:, None, :],
        We1[:, :hid, :], We1[:, hid:2 * hid, :], We1[:, 2 * hid:, :],
        be1[:, None, :], We2, be2[:, None, :],
        Wn1[:, :hid, :], Wn1[:, hid:, :], bn1[:, None, :],
        Wn2, bn2[:, None, :], Wout, bout.reshape(1, code), scale,
    )
    out = pl.pallas_call(
        _gnn_kernel,
        grid=(bsz,),
        in_specs=[pl.BlockSpec((1, ngrid, code), lambda bb: (bb, 0, 0))]
                 + [full(a) for a in args],
        out_specs=pl.BlockSpec((1, ngrid, code), lambda bb: (bb, 0, 0)),
        out_shape=jax.ShapeDtypeStruct((bsz, ngrid, code), jnp.float32),
        scratch_shapes=[
            pltpu.VMEM((2 * _PAD + ngrid, hid), jnp.float32),
            pltpu.VMEM((2 * _PAD + ngrid, hid), jnp.float32),
            pltpu.VMEM((2 * _PAD + ngrid, hid), jnp.float32),
        ],
        compiler_params=pltpu.CompilerParams(
            dimension_semantics=("parallel",)),
    )(y, *args)
    return out


# speedup vs baseline: 30.2666x; 1.3086x over previous
import numpy as np
import jax
import jax.numpy as jnp
from jax.experimental import pallas as pl
from jax.experimental.pallas import tpu as pltpu

_GRID = 8
_SPACING = 2.0
_RADIUS = 3.5
_NGRID = _GRID ** 3
_NP = _NGRID // 2
_PAD = 40


def _stencil_constants():
    r = int(np.floor(_RADIUS / _SPACING)) + 1
    offsets = []
    for dx in range(-r, r + 1):
        for dy in range(-r, r + 1):
            for dz in range(-r, r + 1):
                if dx == 0 and dy == 0 and dz == 0:
                    continue
                d = _SPACING * np.sqrt(dx * dx + dy * dy + dz * dz)
                if d <= _RADIUS + 1e-8:
                    offsets.append((dx, dy, dz, float(d)))
    idx = np.arange(_NGRID)
    ix, iy, iz = idx // (_GRID * _GRID), (idx // _GRID) % _GRID, idx % _GRID
    valid = np.zeros((len(offsets), _NGRID), np.float32)
    for i, (dx, dy, dz, _) in enumerate(offsets):
        valid[i] = ((ix + dx >= 0) & (ix + dx < _GRID) &
                    (iy + dy >= 0) & (iy + dy < _GRID) &
                    (iz + dz >= 0) & (iz + dz < _GRID)).astype(np.float32)
    denom = np.maximum(valid.sum(0), 1.0)
    scale = valid / denom[None, :]
    scale_p = np.repeat(scale.reshape(len(offsets) * _NP, 2), 64, axis=1)
    return offsets, scale_p


_OFFSETS, _SCALE_P = _stencil_constants()
_NOFF = len(_OFFSETS)
_DISTS = sorted({d for (_, _, _, d) in _OFFSETS})


def _bd(w):
    k, m = w.shape[-2], w.shape[-1]
    z = jnp.zeros(w.shape[:-2] + (k, m), w.dtype)
    top = jnp.concatenate([w, z], axis=-1)
    bot = jnp.concatenate([z, w], axis=-1)
    return jnp.concatenate([top, bot], axis=-2)


def _tile2(v):
    return jnp.concatenate([v, v], axis=-1)


def _silu(v):
    u = 0.5 * v
    return u * jnp.tanh(u) + u


def _gnn_kernel(y_ref, win_ref, bin_ref, lng_ref, lnb_ref,
                we1a_ref, we1b_ref, we1c_ref, be1_ref, we2_ref, be2_ref,
                wn1a_ref, wn1b_ref, bn1_ref, wn2_ref, bn2_ref,
                wout_ref, bout_ref, scale_ref, jmean_ref, out_ref,
                az0_ref, azp_ref, azm_ref):
    layers = lng_ref.shape[0]
    n = _NP
    f32 = jnp.float32
    bf = jnp.bfloat16
    az0_ref[0:_PAD, :] = jnp.zeros((_PAD, 128), f32)
    az0_ref[_PAD + n:, :] = jnp.zeros_like(az0_ref[_PAD + n:, :])
    azp_ref[...] = jnp.zeros_like(azp_ref)
    azm_ref[...] = jnp.zeros_like(azm_ref)

    h = jnp.dot(y_ref[0].astype(bf), win_ref[...],
                preferred_element_type=f32) + bin_ref[...]
    for l in range(layers):
        mu = jnp.dot(h, jmean_ref[...], preferred_element_type=f32)
        d = h - mu
        var = jnp.dot(d * d, jmean_ref[...], preferred_element_type=f32)
        hn = d * jax.lax.rsqrt(var + 1e-5) * lng_ref[l] + lnb_ref[l]
        hnb = hn.astype(bf)
        a = jnp.dot(hnb, we1a_ref[l], preferred_element_type=f32)
        b = jnp.dot(hnb, we1b_ref[l], preferred_element_type=f32)
        az0_ref[_PAD:_PAD + n, :] = a
        azp_ref[_PAD:_PAD + n, :] = jnp.concatenate(
            [az0_ref[_PAD:_PAD + n, 64:128],
             az0_ref[_PAD + 1:_PAD + n + 1, 0:64]], axis=1)
        azm_ref[_PAD:_PAD + n, :] = jnp.concatenate(
            [az0_ref[_PAD - 1:_PAD + n - 1, 64:128],
             az0_ref[_PAD:_PAD + n, 0:64]], axis=1)
        bc = {d_: b + (d_ * we1c_ref[l] + be1_ref[l]) for d_ in _DISTS}
        m1s = []
        for (dx, dy, dz, dist) in _OFFSETS:
            base = azp_ref if dz == 1 else (azm_ref if dz == -1 else az0_ref)
            kk = _PAD + dx * 32 + dy * 4
            m1s.append(_silu(base[kk:kk + n, :] + bc[dist]).astype(bf))
        m1 = jnp.concatenate(m1s, axis=0)
        m2 = _silu(jnp.dot(m1, we2_ref[l],
                           preferred_element_type=f32) + be2_ref[l])
        magg = jnp.zeros((n, 128), f32)
        for i in range(_NOFF):
            sl = slice(i * n, (i + 1) * n)
            magg = magg + m2[sl] * scale_ref[sl]
        nd = _silu(jnp.dot(hnb, wn1a_ref[l], preferred_element_type=f32)
                   + jnp.dot(magg.astype(bf), wn1b_ref[l],
                             preferred_element_type=f32)
                   + bn1_ref[l])
        h = hn + jnp.dot(nd.astype(bf), wn2_ref[l],
                         preferred_element_type=f32) + bn2_ref[l]
    out_ref[0] = jnp.dot(h.astype(bf), wout_ref[...],
                         preferred_element_type=f32) + bout_ref[...]


def kernel(y, Win, b_in, ln_g, ln_b, We1, be1, We2, be2,
           Wn1, bn1, Wn2, bn2, Wout, bout):
    bsz, ngrid, code = y.shape
    hid = Win.shape[1]
    bf = jnp.bfloat16

    scale = jnp.asarray(_SCALE_P)
    jmean = _bd(jnp.full((hid, hid), 1.0 / hid, jnp.float32))
    y_p = y.reshape(bsz, ngrid // 2, 2 * code)

    full = lambda arr: pl.BlockSpec(arr.shape, lambda bb: (0,) * arr.ndim)
    args = (
        _bd(Win.astype(bf)), _tile2(b_in.reshape(1, hid)),
        _tile2(ln_g[:, None, :]), _tile2(ln_b[:, None, :]),
        _bd(We1[:, :hid, :].astype(bf)),
        _bd(We1[:, hid:2 * hid, :].astype(bf)),
        _tile2(We1[:, 2 * hid:, :]), _tile2(be1[:, None, :]),
        _bd(We2.astype(bf)), _tile2(be2[:, None, :]),
        _bd(Wn1[:, :hid, :].astype(bf)), _bd(Wn1[:, hid:, :].astype(bf)),
        _tile2(bn1[:, None, :]), _bd(Wn2.astype(bf)),
        _tile2(bn2[:, None, :]), _bd(Wout.astype(bf)),
        _tile2(bout.reshape(1, code)), scale, jmean,
    )
    out = pl.pallas_call(
        _gnn_kernel,
        grid=(bsz,),
        in_specs=[pl.BlockSpec((1, ngrid // 2, 2 * code),
                               lambda bb: (bb, 0, 0))]
                 + [full(a) for a in args],
        out_specs=pl.BlockSpec((1, ngrid // 2, 2 * code),
                               lambda bb: (bb, 0, 0)),
        out_shape=jax.ShapeDtypeStruct((bsz, ngrid // 2, 2 * code),
                                       jnp.float32),
        scratch_shapes=[
            pltpu.VMEM((2 * _PAD + ngrid // 2, 2 * hid), jnp.float32),
            pltpu.VMEM((2 * _PAD + ngrid // 2, 2 * hid), jnp.float32),
            pltpu.VMEM((2 * _PAD + ngrid // 2, 2 * hid), jnp.float32),
        ],
        compiler_params=pltpu.CompilerParams(
            dimension_semantics=("parallel",)),
    )(y_p, *args)
    return out.reshape(bsz, ngrid, code)


# speedup vs baseline: 35.2561x; 1.1649x over previous
import numpy as np
import jax
import jax.numpy as jnp
from jax.experimental import pallas as pl
from jax.experimental.pallas import tpu as pltpu

_GRID = 8
_SPACING = 2.0
_RADIUS = 3.5
_NGRID = _GRID ** 3
_NP = _NGRID // 2
_PAD = 40
_SEC = 2 * _PAD + _NP
_GPB = 2


def _stencil_constants():
    r = int(np.floor(_RADIUS / _SPACING)) + 1
    offsets = []
    for dx in range(-r, r + 1):
        for dy in range(-r, r + 1):
            for dz in range(-r, r + 1):
                if dx == 0 and dy == 0 and dz == 0:
                    continue
                d = _SPACING * np.sqrt(dx * dx + dy * dy + dz * dz)
                if d <= _RADIUS + 1e-8:
                    offsets.append((dx, dy, dz, float(d)))
    idx = np.arange(_NGRID)
    ix, iy, iz = idx // (_GRID * _GRID), (idx // _GRID) % _GRID, idx % _GRID
    valid = np.zeros((len(offsets), _NGRID), np.float32)
    for i, (dx, dy, dz, _) in enumerate(offsets):
        valid[i] = ((ix + dx >= 0) & (ix + dx < _GRID) &
                    (iy + dy >= 0) & (iy + dy < _GRID) &
                    (iz + dz >= 0) & (iz + dz < _GRID)).astype(np.float32)
    denom = np.maximum(valid.sum(0), 1.0)
    scale = valid / denom[None, :]
    sp = np.repeat(scale.reshape(len(offsets), _NP, 2), 64, axis=2)
    sp = np.tile(sp, (1, _GPB, 1)).reshape(len(offsets) * _GPB * _NP, 128)
    return offsets, sp


_OFFSETS, _SCALE_P = _stencil_constants()
_NOFF = len(_OFFSETS)
_DISTS = sorted({d for (_, _, _, d) in _OFFSETS})


def _bd(w):
    k, m = w.shape[-2], w.shape[-1]
    z = jnp.zeros(w.shape[:-2] + (k, m), w.dtype)
    top = jnp.concatenate([w, z], axis=-1)
    bot = jnp.concatenate([z, w], axis=-1)
    return jnp.concatenate([top, bot], axis=-2)


def _tile2(v):
    return jnp.concatenate([v, v], axis=-1)


def _silu(v):
    u = 0.5 * v
    return u * jnp.tanh(u) + u


def _gnn_kernel(y_ref, win_ref, bin_ref, lng_ref, lnb_ref,
                we1a_ref, we1b_ref, we1c_ref, be1_ref, we2_ref, be2_ref,
                wn1a_ref, wn1b_ref, bn1_ref, wn2_ref, bn2_ref,
                wout_ref, bout_ref, scale_ref, jmean_ref, out_ref,
                az0_ref, azp_ref, azm_ref):
    layers = lng_ref.shape[0]
    n = _NP
    rows = _GPB * n
    f32 = jnp.float32
    bf = jnp.bfloat16
    az0_ref[...] = jnp.zeros_like(az0_ref)
    azp_ref[...] = jnp.zeros_like(azp_ref)
    azm_ref[...] = jnp.zeros_like(azm_ref)

    h = jnp.concatenate(
        [jnp.dot(y_ref[g].astype(bf), win_ref[...],
                 preferred_element_type=f32) for g in range(_GPB)],
        axis=0) + bin_ref[...]
    for l in range(layers):
        mu = jnp.dot(h, jmean_ref[...], preferred_element_type=f32)
        d = h - mu
        var = jnp.dot(d * d, jmean_ref[...], preferred_element_type=f32)
        hn = d * jax.lax.rsqrt(var + 1e-5) * lng_ref[l] + lnb_ref[l]
        hnb = hn.astype(bf)
        a = jnp.dot(hnb, we1a_ref[l], preferred_element_type=f32)
        b = jnp.dot(hnb, we1b_ref[l], preferred_element_type=f32)
        for g in range(_GPB):
            o = g * _SEC + _PAD
            az0_ref[o:o + n, :] = a[g * n:(g + 1) * n]
        for g in range(_GPB):
            o = g * _SEC + _PAD
            azp_ref[o:o + n, :] = jnp.concatenate(
                [az0_ref[o:o + n, 64:128],
                 az0_ref[o + 1:o + n + 1, 0:64]], axis=1)
            azm_ref[o:o + n, :] = jnp.concatenate(
                [az0_ref[o - 1:o + n - 1, 64:128],
                 az0_ref[o:o + n, 0:64]], axis=1)
        bc = {d_: b + (d_ * we1c_ref[l] + be1_ref[l]) for d_ in _DISTS}
        m1s = []
        for (dx, dy, dz, dist) in _OFFSETS:
            base = azp_ref if dz == 1 else (azm_ref if dz == -1 else az0_ref)
            kk = _PAD + dx * 32 + dy * 4
            sh = jnp.concatenate(
                [base[g * _SEC + kk:g * _SEC + kk + n, :]
                 for g in range(_GPB)], axis=0)
            m1s.append(_silu(sh + bc[dist]).astype(bf))
        m1 = jnp.concatenate(m1s, axis=0)
        m2 = _silu(jnp.dot(m1, we2_ref[l],
                           preferred_element_type=f32) + be2_ref[l])
        magg = jnp.zeros((rows, 128), f32)
        for i in range(_NOFF):
            sl = slice(i * rows, (i + 1) * rows)
            magg = magg + m2[sl] * scale_ref[sl]
        nd = _silu(jnp.dot(hnb, wn1a_ref[l], preferred_element_type=f32)
                   + jnp.dot(magg.astype(bf), wn1b_ref[l],
                             preferred_element_type=f32)
                   + bn1_ref[l])
        h = hn + jnp.dot(nd.astype(bf), wn2_ref[l],
                         preferred_element_type=f32) + bn2_ref[l]
    z = jnp.dot(h.astype(bf), wout_ref[...],
                preferred_element_type=f32) + bout_ref[...]
    for g in range(_GPB):
        out_ref[g] = z[g * n:(g + 1) * n]


def kernel(y, Win, b_in, ln_g, ln_b, We1, be1, We2, be2,
           Wn1, bn1, Wn2, bn2, Wout, bout):
    bsz, ngrid, code = y.shape
    hid = Win.shape[1]
    bf = jnp.bfloat16

    scale = jnp.asarray(_SCALE_P)
    jmean = _bd(jnp.full((hid, hid), 1.0 / hid, jnp.float32))
    y_p = y.reshape(bsz, ngrid // 2, 2 * code)

    full = lambda arr: pl.BlockSpec(arr.shape, lambda bb: (0,) * arr.ndim)
    args = (
        _bd(Win.astype(bf)), _tile2(b_in.reshape(1, hid)),
        _tile2(ln_g[:, None, :]), _tile2(ln_b[:, None, :]),
        _bd(We1[:, :hid, :].astype(bf)),
        _bd(We1[:, hid:2 * hid, :].astype(bf)),
        _tile2(We1[:, 2 * hid:, :]), _tile2(be1[:, None, :]),
        _bd(We2.astype(bf)), _tile2(be2[:, None, :]),
        _bd(Wn1[:, :hid, :].astype(bf)), _bd(Wn1[:, hid:, :].astype(bf)),
        _tile2(bn1[:, None, :]), _bd(Wn2.astype(bf)),
        _tile2(bn2[:, None, :]), _bd(Wout.astype(bf)),
        _tile2(bout.reshape(1, code)), scale, jmean,
    )
    out = pl.pallas_call(
        _gnn_kernel,
        grid=(bsz // _GPB,),
        in_specs=[pl.BlockSpec((_GPB, ngrid // 2, 2 * code),
                               lambda bb: (bb, 0, 0))]
                 + [full(a) for a in args],
        out_specs=pl.BlockSpec((_GPB, ngrid // 2, 2 * code),
                               lambda bb: (bb, 0, 0)),
        out_shape=jax.ShapeDtypeStruct((bsz, ngrid // 2, 2 * code),
                                       jnp.float32),
        scratch_shapes=[
            pltpu.VMEM((_GPB * _SEC, 2 * hid), jnp.float32),
            pltpu.VMEM((_GPB * _SEC, 2 * hid), jnp.float32),
            pltpu.VMEM((_GPB * _SEC, 2 * hid), jnp.float32),
        ],
        compiler_params=pltpu.CompilerParams(
            dimension_semantics=("parallel",)),
    )(y_p, *args)
    return out.reshape(bsz, ngrid, code)
